# Initial kernel scaffold; baseline (speedup 1.0000x reference)
import jax
import jax.numpy as jnp
from jax.experimental import pallas as pl


def kernel(x, edge_index, W1, b1, W2, b2, W3, b3, Wc, bc):
    raise NotImplementedError("write your pallas kernel here")


# speedup vs baseline: 16.9149x; 16.9149x over previous
import functools

import jax
import jax.numpy as jnp
from jax import lax
from jax.experimental import pallas as pl
from jax.experimental.pallas import tpu as pltpu
from jax.experimental.pallas import tpu_sc as plsc

N = 10000
E = 320000
D = 128
H = 64
H3 = 32
C = 4

NC = 2
NS = 16
NW = NC * NS
CHUNK = 128
EPT = 10240
CHUNKS = EPT // CHUNK
NPAD = 10240
RPT = NPAD // NS

_f32 = jnp.float32


def _sc_mesh():
    return plsc.VectorSubcoreMesh(core_axis_name="c", subcore_axis_name="s")


@functools.partial(
    pl.kernel,
    out_type=jax.ShapeDtypeStruct((NC, NPAD), _f32),
    mesh=_sc_mesh(),
    scratch_types=[
        pltpu.VMEM((CHUNKS, CHUNK), jnp.int32),
        pltpu.VMEM((CHUNK,), _f32),
        pltpu.VMEM_SHARED((NPAD,), _f32),
    ],
)
def _deg_kernel(dst_hbm, ones_hbm, zeros_hbm, out_hbm, dst_v, ones_v, deg_sh):
    c = lax.axis_index("c")
    s = lax.axis_index("s")
    wid = c * NS + s
    pltpu.sync_copy(zeros_hbm.at[pl.ds(s * RPT, RPT)], deg_sh.at[pl.ds(s * RPT, RPT)])
    pltpu.sync_copy(dst_hbm.at[wid], dst_v)
    pltpu.sync_copy(ones_hbm, ones_v)
    plsc.subcore_barrier()

    def body(j, carry):
        pltpu.sync_copy(ones_v, deg_sh.at[dst_v.at[j]], add=True)
        return carry

    lax.fori_loop(0, CHUNKS, body, 0)
    plsc.subcore_barrier()
    pltpu.sync_copy(deg_sh.at[pl.ds(s * RPT, RPT)], out_hbm.at[c, pl.ds(s * RPT, RPT)])


def _make_agg_kernel(h):
    @functools.partial(
        pl.kernel,
        out_type=jax.ShapeDtypeStruct((NC, NPAD, h), _f32),
        mesh=_sc_mesh(),
        compiler_params=pltpu.CompilerParams(use_tc_tiling_on_sc=False),
        scratch_types=[
            pltpu.VMEM((CHUNKS, CHUNK), jnp.int32),
            pltpu.VMEM((CHUNKS, CHUNK), jnp.int32),
            pltpu.VMEM((CHUNK, h), _f32),
            pltpu.VMEM((CHUNK, h), _f32),
            pltpu.VMEM_SHARED((NPAD, h), _f32),
            pltpu.SemaphoreType.DMA,
            pltpu.SemaphoreType.DMA,
        ],
    )
    def agg_kernel(src_hbm, dst_hbm, hs_hbm, zeros_hbm, out_hbm,
                   src_v, dst_v, msg0_v, msg1_v, agg_sh, sem0, sem1):
        c = lax.axis_index("c")
        s = lax.axis_index("s")
        wid = c * NS + s
        pltpu.sync_copy(zeros_hbm.at[pl.ds(s * RPT, RPT)],
                        agg_sh.at[pl.ds(s * RPT, RPT)])
        pltpu.sync_copy(src_hbm.at[wid], src_v)
        pltpu.sync_copy(dst_hbm.at[wid], dst_v)
        plsc.subcore_barrier()

        pltpu.async_copy(hs_hbm.at[src_v.at[0]], msg0_v, sem0)

        def body(i, carry):
            j = 2 * i
            pltpu.async_copy(hs_hbm.at[src_v.at[j + 1]], msg1_v, sem1)
            pltpu.make_async_copy(hs_hbm.at[src_v.at[j]], msg0_v, sem0).wait()
            pltpu.sync_copy(msg0_v, agg_sh.at[dst_v.at[j]], add=True)

            @pl.when(i < CHUNKS // 2 - 1)
            def _():
                pltpu.async_copy(hs_hbm.at[src_v.at[j + 2]], msg0_v, sem0)

            pltpu.make_async_copy(hs_hbm.at[src_v.at[j + 1]], msg1_v, sem1).wait()
            pltpu.sync_copy(msg1_v, agg_sh.at[dst_v.at[j + 1]], add=True)
            return carry

        lax.fori_loop(0, CHUNKS // 2, body, 0)
        plsc.subcore_barrier()
        pltpu.sync_copy(agg_sh.at[pl.ds(s * RPT, RPT)],
                        out_hbm.at[c, pl.ds(s * RPT, RPT)])

    return agg_kernel


_agg64 = _make_agg_kernel(H)
_agg32 = _make_agg_kernel(H3)


def _mm_body(x_ref, w_ref, o_ref):
    o_ref[...] = jnp.dot(x_ref[...], w_ref[...], preferred_element_type=_f32)


def _dinv_body(dega_ref, degb_ref, h_ref, dinv_ref, hs_ref):
    deg = dega_ref[...] + degb_ref[...] + 1.0
    dinv = lax.rsqrt(deg)
    dinv_ref[...] = dinv
    hs_ref[...] = h_ref[...] * dinv


def _mid_body(agga_ref, aggb_ref, hs_ref, dinv_ref, b_ref, w_ref, o_ref):
    dinv = dinv_ref[...]
    hcur = jnp.maximum(
        dinv * (agga_ref[...] + aggb_ref[...] + hs_ref[...]) + b_ref[...], 0.0)
    o_ref[...] = jnp.dot(hcur, w_ref[...], preferred_element_type=_f32) * dinv


def _final_body(agga_ref, aggb_ref, hs_ref, dinv_ref, b_ref, wc_ref, bc_ref,
                h3_ref, z_ref):
    dinv = dinv_ref[...]
    h3 = jnp.maximum(
        dinv * (agga_ref[...] + aggb_ref[...] + hs_ref[...]) + b_ref[...], 0.0)
    h3_ref[...] = h3
    z_ref[...] = jnp.dot(h3, wc_ref[...], preferred_element_type=_f32) + bc_ref[...]


def _sds(shape):
    return jax.ShapeDtypeStruct(shape, _f32)


def kernel(x, edge_index, W1, b1, W2, b2, W3, b3, Wc, bc):
    src = edge_index[0]
    dst = edge_index[1]
    pad = EPT * NW - E
    srcp = jnp.concatenate([src, jnp.zeros((pad,), src.dtype)]).reshape(NW, CHUNKS, CHUNK)
    dstp = jnp.concatenate([dst, jnp.full((pad,), N, dst.dtype)]).reshape(NW, CHUNKS, CHUNK)
    ones = jnp.ones((CHUNK,), _f32)
    z1 = jnp.zeros((NPAD,), _f32)
    z64 = jnp.zeros((NPAD, H), _f32)
    z32 = jnp.zeros((NPAD, H3), _f32)

    h1_raw = pl.pallas_call(_mm_body, out_shape=_sds((N, H)))(x, W1)
    degp = _deg_kernel(dstp, ones, z1)

    dinv, hs1 = pl.pallas_call(_dinv_body, out_shape=(_sds((N, 1)), _sds((N, H))))(
        degp[0, :N, None], degp[1, :N, None], h1_raw)

    agg1 = _agg64(srcp, dstp, hs1, z64)
    hs2 = pl.pallas_call(_mid_body, out_shape=_sds((N, H)))(
        agg1[0, :N], agg1[1, :N], hs1, dinv, b1.reshape(1, H), W2)

    agg2 = _agg64(srcp, dstp, hs2, z64)
    hs3 = pl.pallas_call(_mid_body, out_shape=_sds((N, H3)))(
        agg2[0, :N], agg2[1, :N], hs2, dinv, b2.reshape(1, H), W3)

    agg3 = _agg32(srcp, dstp, hs3, z32)
    h3, z = pl.pallas_call(_final_body, out_shape=(_sds((N, H3)), _sds((N, C))))(
        agg3[0, :N], agg3[1, :N], hs3, dinv, b3.reshape(1, H3), Wc, bc.reshape(1, C))
    return (h3, z)


# speedup vs baseline: 17.9288x; 1.0599x over previous
import functools

import jax
import jax.numpy as jnp
from jax import lax
from jax.experimental import pallas as pl
from jax.experimental.pallas import tpu as pltpu
from jax.experimental.pallas import tpu_sc as plsc

N = 10000
E = 320000
D = 128
H = 64
H3 = 32
C = 4

NC = 2
NS = 16
NW = NC * NS
CHUNK = 128
EPT = 10240
CHUNKS = EPT // CHUNK
NPAD = 10240
RPT = NPAD // NS

K0 = 116
K1 = 44

_f32 = jnp.float32


def _sc_mesh():
    return plsc.VectorSubcoreMesh(core_axis_name="c", subcore_axis_name="s")


@functools.partial(
    pl.kernel,
    out_type=jax.ShapeDtypeStruct((NC, NPAD), _f32),
    mesh=_sc_mesh(),
    scratch_types=[
        pltpu.VMEM((CHUNKS, CHUNK), jnp.int32),
        pltpu.VMEM((CHUNK,), _f32),
        pltpu.VMEM_SHARED((NPAD,), _f32),
    ],
)
def _deg_kernel(dst_hbm, ones_hbm, zeros_hbm, out_hbm, dst_v, ones_v, deg_sh):
    c = lax.axis_index("c")
    s = lax.axis_index("s")
    wid = c * NS + s
    pltpu.sync_copy(zeros_hbm.at[pl.ds(s * RPT, RPT)], deg_sh.at[pl.ds(s * RPT, RPT)])
    pltpu.sync_copy(dst_hbm.at[wid], dst_v)
    pltpu.sync_copy(ones_hbm, ones_v)
    plsc.subcore_barrier()

    def body(j, carry):
        pltpu.sync_copy(ones_v, deg_sh.at[dst_v.at[j]], add=True)
        return carry

    lax.fori_loop(0, CHUNKS, body, 0)
    plsc.subcore_barrier()
    pltpu.sync_copy(deg_sh.at[pl.ds(s * RPT, RPT)], out_hbm.at[c, pl.ds(s * RPT, RPT)])


def _make_agg_kernel(h):
    @functools.partial(
        pl.kernel,
        out_type=jax.ShapeDtypeStruct((NC, NPAD, h), _f32),
        mesh=_sc_mesh(),
        compiler_params=pltpu.CompilerParams(use_tc_tiling_on_sc=False),
        scratch_types=[
            pltpu.VMEM((K0, CHUNK), jnp.int32),
            pltpu.VMEM((K0, CHUNK), jnp.int32),
            pltpu.VMEM((CHUNK, h), _f32),
            pltpu.VMEM((CHUNK, h), _f32),
            pltpu.VMEM_SHARED((NPAD, h), _f32),
            pltpu.SemaphoreType.DMA,
            pltpu.SemaphoreType.DMA,
        ],
    )
    def agg_kernel(src_hbm, dst_hbm, hs_hbm, zeros_hbm, out_hbm,
                   src_v, dst_v, msg0_v, msg1_v, agg_sh, sem0, sem1):
        c = lax.axis_index("c")
        s = lax.axis_index("s")
        wid = c * NS + s
        pltpu.sync_copy(zeros_hbm.at[pl.ds(s * RPT, RPT)],
                        agg_sh.at[pl.ds(s * RPT, RPT)])
        pltpu.sync_copy(src_hbm.at[wid], src_v)
        pltpu.sync_copy(dst_hbm.at[wid], dst_v)
        plsc.subcore_barrier()

        def run_pipe(nchunks):
            pltpu.async_copy(hs_hbm.at[src_v.at[0]], msg0_v, sem0)

            def body(i, carry):
                j = 2 * i
                pltpu.async_copy(hs_hbm.at[src_v.at[j + 1]], msg1_v, sem1)
                pltpu.make_async_copy(hs_hbm.at[src_v.at[j]], msg0_v, sem0).wait()
                pltpu.sync_copy(msg0_v, agg_sh.at[dst_v.at[j]], add=True)

                @pl.when(i < nchunks // 2 - 1)
                def _():
                    pltpu.async_copy(hs_hbm.at[src_v.at[j + 2]], msg0_v, sem0)

                pltpu.make_async_copy(hs_hbm.at[src_v.at[j + 1]], msg1_v, sem1).wait()
                pltpu.sync_copy(msg1_v, agg_sh.at[dst_v.at[j + 1]], add=True)
                return carry

            lax.fori_loop(0, nchunks // 2, body, 0)

        @pl.when(c == 0)
        def _():
            run_pipe(K0)

        @pl.when(c == 1)
        def _():
            run_pipe(K1)

        plsc.subcore_barrier()
        pltpu.sync_copy(agg_sh.at[pl.ds(s * RPT, RPT)],
                        out_hbm.at[c, pl.ds(s * RPT, RPT)])

    return agg_kernel


_agg64 = _make_agg_kernel(H)
_agg32 = _make_agg_kernel(H3)


def _mm_body(x_ref, w_ref, o_ref):
    o_ref[...] = jnp.dot(x_ref[...], w_ref[...], preferred_element_type=_f32)


def _dinv_body(dega_ref, degb_ref, h_ref, dinv_ref, hs_ref):
    deg = dega_ref[...] + degb_ref[...] + 1.0
    dinv = lax.rsqrt(deg)
    dinv_ref[...] = dinv
    hs_ref[...] = h_ref[...] * dinv


def _mid_body(agga_ref, aggb_ref, hs_ref, dinv_ref, b_ref, w_ref, o_ref):
    dinv = dinv_ref[...]
    hcur = jnp.maximum(
        dinv * (agga_ref[...] + aggb_ref[...] + hs_ref[...]) + b_ref[...], 0.0)
    o_ref[...] = jnp.dot(hcur, w_ref[...], preferred_element_type=_f32) * dinv


def _final_body(agga_ref, aggb_ref, hs_ref, dinv_ref, b_ref, wc_ref, bc_ref,
                h3_ref, z_ref):
    dinv = dinv_ref[...]
    h3 = jnp.maximum(
        dinv * (agga_ref[...] + aggb_ref[...] + hs_ref[...]) + b_ref[...], 0.0)
    h3_ref[...] = h3
    z_ref[...] = jnp.dot(h3, wc_ref[...], preferred_element_type=_f32) + bc_ref[...]


def _sds(shape):
    return jax.ShapeDtypeStruct(shape, _f32)


def kernel(x, edge_index, W1, b1, W2, b2, W3, b3, Wc, bc):
    src = edge_index[0]
    dst = edge_index[1]
    pad = EPT * NW - E
    srcf = jnp.concatenate([src, jnp.zeros((pad,), src.dtype)])
    dstf = jnp.concatenate([dst, jnp.full((pad,), N, dst.dtype)])
    dstp = dstf.reshape(NW, CHUNKS, CHUNK)

    def _split(flat):
        n0 = NS * K0 * CHUNK
        a = flat[:n0].reshape(NS, K0, CHUNK)
        b = flat[n0:].reshape(NS, K1, CHUNK)
        b = jnp.pad(b, ((0, 0), (0, K0 - K1), (0, 0)))
        return jnp.concatenate([a, b], axis=0)

    srcp_a = _split(srcf)
    dstp_a = _split(dstf)
    ones = jnp.ones((CHUNK,), _f32)
    z1 = jnp.zeros((NPAD,), _f32)
    z64 = jnp.zeros((NPAD, H), _f32)
    z32 = jnp.zeros((NPAD, H3), _f32)

    h1_raw = pl.pallas_call(_mm_body, out_shape=_sds((N, H)))(x, W1)
    degp = _deg_kernel(dstp, ones, z1)

    dinv, hs1 = pl.pallas_call(_dinv_body, out_shape=(_sds((N, 1)), _sds((N, H))))(
        degp[0, :N, None], degp[1, :N, None], h1_raw)

    agg1 = _agg64(srcp_a, dstp_a, hs1, z64)
    hs2 = pl.pallas_call(_mid_body, out_shape=_sds((N, H)))(
        agg1[0, :N], agg1[1, :N], hs1, dinv, b1.reshape(1, H), W2)

    agg2 = _agg64(srcp_a, dstp_a, hs2, z64)
    hs3 = pl.pallas_call(_mid_body, out_shape=_sds((N, H3)))(
        agg2[0, :N], agg2[1, :N], hs2, dinv, b2.reshape(1, H), W3)

    agg3 = _agg32(srcp_a, dstp_a, hs3, z32)
    h3, z = pl.pallas_call(_final_body, out_shape=(_sds((N, H3)), _sds((N, C))))(
        agg3[0, :N], agg3[1, :N], hs3, dinv, b3.reshape(1, H3), Wc, bc.reshape(1, C))
    return (h3, z)


# speedup vs baseline: 27.7383x; 1.5471x over previous
import functools

import jax
import jax.numpy as jnp
from jax import lax
from jax.experimental import pallas as pl
from jax.experimental.pallas import tpu as pltpu
from jax.experimental.pallas import tpu_sc as plsc

N = 10000
E = 320000
D = 128
H = 64
H3 = 32
C = 4

NC = 2
NS = 16
NW = NC * NS
CHUNK = 128
EPT = 10240
CHUNKS = EPT // CHUNK
NPAD = 10240
RPT = NPAD // NS

K0 = 116
K1 = 44

_f32 = jnp.float32


def _sc_mesh():
    return plsc.VectorSubcoreMesh(core_axis_name="c", subcore_axis_name="s")


@functools.partial(
    pl.kernel,
    out_type=jax.ShapeDtypeStruct((NC, NPAD), _f32),
    mesh=_sc_mesh(),
    scratch_types=[
        pltpu.VMEM((CHUNKS, CHUNK), jnp.int32),
        pltpu.VMEM((CHUNK,), _f32),
        pltpu.VMEM_SHARED((NPAD,), _f32),
    ],
)
def _deg_kernel(dst_hbm, ones_hbm, zeros_hbm, out_hbm, dst_v, ones_v, deg_sh):
    c = lax.axis_index("c")
    s = lax.axis_index("s")
    wid = c * NS + s
    pltpu.sync_copy(zeros_hbm.at[pl.ds(s * RPT, RPT)], deg_sh.at[pl.ds(s * RPT, RPT)])
    pltpu.sync_copy(dst_hbm.at[wid], dst_v)
    pltpu.sync_copy(ones_hbm, ones_v)
    plsc.subcore_barrier()

    def body(j, carry):
        pltpu.sync_copy(ones_v, deg_sh.at[dst_v.at[j]], add=True)
        return carry

    lax.fori_loop(0, CHUNKS, body, 0)
    plsc.subcore_barrier()
    pltpu.sync_copy(deg_sh.at[pl.ds(s * RPT, RPT)], out_hbm.at[c, pl.ds(s * RPT, RPT)])


def _make_agg_kernel(h):
    @functools.partial(
        pl.kernel,
        out_type=jax.ShapeDtypeStruct((NC, NPAD, h), _f32),
        mesh=_sc_mesh(),
        compiler_params=pltpu.CompilerParams(use_tc_tiling_on_sc=False),
        scratch_types=[
            pltpu.VMEM((K0, CHUNK), jnp.int32),
            pltpu.VMEM((K0, CHUNK), jnp.int32),
            pltpu.VMEM((CHUNK, h), _f32),
            pltpu.VMEM((CHUNK, h), _f32),
            pltpu.VMEM_SHARED((NPAD, h), _f32),
            pltpu.VMEM_SHARED((N, h), _f32),
            pltpu.SemaphoreType.DMA,
            pltpu.SemaphoreType.DMA,
        ],
    )
    def agg_kernel(src_hbm, dst_hbm, hs_hbm, zeros_hbm, out_hbm,
                   src_v, dst_v, msg0_v, msg1_v, agg_sh, hs_sh, sem0, sem1):
        c = lax.axis_index("c")
        s = lax.axis_index("s")
        wid = c * NS + s
        pltpu.sync_copy(zeros_hbm.at[pl.ds(s * RPT, RPT)],
                        agg_sh.at[pl.ds(s * RPT, RPT)])
        pltpu.sync_copy(hs_hbm.at[pl.ds(s * (N // NS), N // NS)],
                        hs_sh.at[pl.ds(s * (N // NS), N // NS)])
        pltpu.sync_copy(src_hbm.at[wid], src_v)
        pltpu.sync_copy(dst_hbm.at[wid], dst_v)
        plsc.subcore_barrier()

        def run_pipe(nchunks):
            pltpu.async_copy(hs_sh.at[src_v.at[0]], msg0_v, sem0)

            def body(i, carry):
                j = 2 * i
                pltpu.async_copy(hs_sh.at[src_v.at[j + 1]], msg1_v, sem1)
                pltpu.make_async_copy(hs_sh.at[src_v.at[j]], msg0_v, sem0).wait()
                pltpu.sync_copy(msg0_v, agg_sh.at[dst_v.at[j]], add=True)

                @pl.when(i < nchunks // 2 - 1)
                def _():
                    pltpu.async_copy(hs_sh.at[src_v.at[j + 2]], msg0_v, sem0)

                pltpu.make_async_copy(hs_sh.at[src_v.at[j + 1]], msg1_v, sem1).wait()
                pltpu.sync_copy(msg1_v, agg_sh.at[dst_v.at[j + 1]], add=True)
                return carry

            lax.fori_loop(0, nchunks // 2, body, 0)

        @pl.when(c == 0)
        def _():
            run_pipe(K0)

        @pl.when(c == 1)
        def _():
            run_pipe(K1)

        plsc.subcore_barrier()
        pltpu.sync_copy(agg_sh.at[pl.ds(s * RPT, RPT)],
                        out_hbm.at[c, pl.ds(s * RPT, RPT)])

    return agg_kernel


_agg64 = _make_agg_kernel(H)
_agg32 = _make_agg_kernel(H3)


def _mm_body(x_ref, w_ref, o_ref):
    o_ref[...] = jnp.dot(x_ref[...], w_ref[...], preferred_element_type=_f32)


def _dinv_body(dega_ref, degb_ref, h_ref, dinv_ref, hs_ref):
    deg = dega_ref[...] + degb_ref[...] + 1.0
    dinv = lax.rsqrt(deg)
    dinv_ref[...] = dinv
    hs_ref[...] = h_ref[...] * dinv


def _mid_body(agga_ref, aggb_ref, hs_ref, dinv_ref, b_ref, w_ref, o_ref):
    dinv = dinv_ref[...]
    hcur = jnp.maximum(
        dinv * (agga_ref[...] + aggb_ref[...] + hs_ref[...]) + b_ref[...], 0.0)
    o_ref[...] = jnp.dot(hcur, w_ref[...], preferred_element_type=_f32) * dinv


def _final_body(agga_ref, aggb_ref, hs_ref, dinv_ref, b_ref, wc_ref, bc_ref,
                h3_ref, z_ref):
    dinv = dinv_ref[...]
    h3 = jnp.maximum(
        dinv * (agga_ref[...] + aggb_ref[...] + hs_ref[...]) + b_ref[...], 0.0)
    h3_ref[...] = h3
    z_ref[...] = jnp.dot(h3, wc_ref[...], preferred_element_type=_f32) + bc_ref[...]


def _sds(shape):
    return jax.ShapeDtypeStruct(shape, _f32)


def kernel(x, edge_index, W1, b1, W2, b2, W3, b3, Wc, bc):
    src = edge_index[0]
    dst = edge_index[1]
    pad = EPT * NW - E
    srcf = jnp.concatenate([src, jnp.zeros((pad,), src.dtype)])
    dstf = jnp.concatenate([dst, jnp.full((pad,), N, dst.dtype)])
    dstp = dstf.reshape(NW, CHUNKS, CHUNK)

    def _split(flat):
        n0 = NS * K0 * CHUNK
        a = flat[:n0].reshape(NS, K0, CHUNK)
        b = flat[n0:].reshape(NS, K1, CHUNK)
        b = jnp.pad(b, ((0, 0), (0, K0 - K1), (0, 0)))
        return jnp.concatenate([a, b], axis=0)

    srcp_a = _split(srcf)
    dstp_a = _split(dstf)
    ones = jnp.ones((CHUNK,), _f32)
    z1 = jnp.zeros((NPAD,), _f32)
    z64 = jnp.zeros((NPAD, H), _f32)
    z32 = jnp.zeros((NPAD, H3), _f32)

    h1_raw = pl.pallas_call(_mm_body, out_shape=_sds((N, H)))(x, W1)
    degp = _deg_kernel(dstp, ones, z1)

    dinv, hs1 = pl.pallas_call(_dinv_body, out_shape=(_sds((N, 1)), _sds((N, H))))(
        degp[0, :N, None], degp[1, :N, None], h1_raw)

    agg1 = _agg64(srcp_a, dstp_a, hs1, z64)
    hs2 = pl.pallas_call(_mid_body, out_shape=_sds((N, H)))(
        agg1[0, :N], agg1[1, :N], hs1, dinv, b1.reshape(1, H), W2)

    agg2 = _agg64(srcp_a, dstp_a, hs2, z64)
    hs3 = pl.pallas_call(_mid_body, out_shape=_sds((N, H3)))(
        agg2[0, :N], agg2[1, :N], hs2, dinv, b2.reshape(1, H), W3)

    agg3 = _agg32(srcp_a, dstp_a, hs3, z32)
    h3, z = pl.pallas_call(_final_body, out_shape=(_sds((N, H3)), _sds((N, C))))(
        agg3[0, :N], agg3[1, :N], hs3, dinv, b3.reshape(1, H3), Wc, bc.reshape(1, C))
    return (h3, z)


# speedup vs baseline: 32.9522x; 1.1880x over previous
import functools

import jax
import jax.numpy as jnp
from jax import lax
from jax.experimental import pallas as pl
from jax.experimental.pallas import tpu as pltpu
from jax.experimental.pallas import tpu_sc as plsc

N = 10000
E = 320000
D = 128
H = 64
H3 = 32
C = 4

NC = 2
NS = 16
NW = NC * NS
CHUNK = 128
EPT = 10240
CHUNKS = EPT // CHUNK
NPAD = 10240
RPT = NPAD // NS

K0 = 80
K1 = 80

_f32 = jnp.float32


def _sc_mesh():
    return plsc.VectorSubcoreMesh(core_axis_name="c", subcore_axis_name="s")


@functools.partial(
    pl.kernel,
    out_type=jax.ShapeDtypeStruct((NC, NPAD), _f32),
    mesh=_sc_mesh(),
    scratch_types=[
        pltpu.VMEM((CHUNKS, CHUNK), jnp.int32),
        pltpu.VMEM((CHUNK,), _f32),
        pltpu.VMEM_SHARED((NPAD,), _f32),
    ],
)
def _deg_kernel(dst_hbm, ones_hbm, zeros_hbm, out_hbm, dst_v, ones_v, deg_sh):
    c = lax.axis_index("c")
    s = lax.axis_index("s")
    wid = c * NS + s
    pltpu.sync_copy(zeros_hbm.at[pl.ds(s * RPT, RPT)], deg_sh.at[pl.ds(s * RPT, RPT)])
    pltpu.sync_copy(dst_hbm.at[wid], dst_v)
    pltpu.sync_copy(ones_hbm, ones_v)
    plsc.subcore_barrier()

    def body(j, carry):
        pltpu.sync_copy(ones_v, deg_sh.at[dst_v.at[j]], add=True)
        return carry

    lax.fori_loop(0, CHUNKS, body, 0)
    plsc.subcore_barrier()
    pltpu.sync_copy(deg_sh.at[pl.ds(s * RPT, RPT)], out_hbm.at[c, pl.ds(s * RPT, RPT)])


def _make_agg_kernel(h):
    @functools.partial(
        pl.kernel,
        out_type=jax.ShapeDtypeStruct((NC, NPAD, h), _f32),
        mesh=_sc_mesh(),
        compiler_params=pltpu.CompilerParams(use_tc_tiling_on_sc=False),
        scratch_types=[
            pltpu.VMEM((K0, CHUNK), jnp.int32),
            pltpu.VMEM((K0, CHUNK), jnp.int32),
            pltpu.VMEM((CHUNK, h), _f32),
            pltpu.VMEM((CHUNK, h), _f32),
            pltpu.VMEM_SHARED((NPAD, h), _f32),
            pltpu.VMEM_SHARED((N, h), _f32),
            pltpu.SemaphoreType.DMA,
            pltpu.SemaphoreType.DMA,
        ],
    )
    def agg_kernel(src_hbm, dst_hbm, hs_hbm, zeros_hbm, out_hbm,
                   src_v, dst_v, msg0_v, msg1_v, agg_sh, hs_sh, sem0, sem1):
        c = lax.axis_index("c")
        s = lax.axis_index("s")
        wid = c * NS + s
        pltpu.sync_copy(zeros_hbm.at[pl.ds(s * RPT, RPT)],
                        agg_sh.at[pl.ds(s * RPT, RPT)])
        pltpu.sync_copy(hs_hbm.at[pl.ds(s * (N // NS), N // NS)],
                        hs_sh.at[pl.ds(s * (N // NS), N // NS)])
        pltpu.sync_copy(src_hbm.at[wid], src_v)
        pltpu.sync_copy(dst_hbm.at[wid], dst_v)
        plsc.subcore_barrier()

        def run_pipe(nchunks):
            pltpu.async_copy(hs_sh.at[src_v.at[0]], msg0_v, sem0)

            def body(i, carry):
                j = 2 * i
                pltpu.async_copy(hs_sh.at[src_v.at[j + 1]], msg1_v, sem1)
                pltpu.make_async_copy(hs_sh.at[src_v.at[j]], msg0_v, sem0).wait()
                pltpu.sync_copy(msg0_v, agg_sh.at[dst_v.at[j]], add=True)

                @pl.when(i < nchunks // 2 - 1)
                def _():
                    pltpu.async_copy(hs_sh.at[src_v.at[j + 2]], msg0_v, sem0)

                pltpu.make_async_copy(hs_sh.at[src_v.at[j + 1]], msg1_v, sem1).wait()
                pltpu.sync_copy(msg1_v, agg_sh.at[dst_v.at[j + 1]], add=True)
                return carry

            lax.fori_loop(0, nchunks // 2, body, 0)

        @pl.when(c == 0)
        def _():
            run_pipe(K0)

        @pl.when(c == 1)
        def _():
            run_pipe(K1)

        plsc.subcore_barrier()
        pltpu.sync_copy(agg_sh.at[pl.ds(s * RPT, RPT)],
                        out_hbm.at[c, pl.ds(s * RPT, RPT)])

    return agg_kernel


_agg64 = _make_agg_kernel(H)
_agg32 = _make_agg_kernel(H3)


def _mm_dinv_body(x_ref, w_ref, dega_ref, degb_ref, dinv_ref, hs_ref):
    deg = dega_ref[...] + degb_ref[...] + 1.0
    dinv = lax.rsqrt(deg)
    dinv_ref[...] = dinv
    h = jnp.dot(x_ref[...], w_ref[...], preferred_element_type=_f32)
    hs_ref[...] = h * dinv


def _mid_body(agga_ref, aggb_ref, hs_ref, dinv_ref, b_ref, w_ref, o_ref):
    dinv = dinv_ref[...]
    hcur = jnp.maximum(
        dinv * (agga_ref[...] + aggb_ref[...] + hs_ref[...]) + b_ref[...], 0.0)
    o_ref[...] = jnp.dot(hcur, w_ref[...], preferred_element_type=_f32) * dinv


def _final_body(agga_ref, aggb_ref, hs_ref, dinv_ref, b_ref, wc_ref, bc_ref,
                h3_ref, z_ref):
    dinv = dinv_ref[...]
    h3 = jnp.maximum(
        dinv * (agga_ref[...] + aggb_ref[...] + hs_ref[...]) + b_ref[...], 0.0)
    h3_ref[...] = h3
    z_ref[...] = jnp.dot(h3, wc_ref[...], preferred_element_type=_f32) + bc_ref[...]


def _sds(shape):
    return jax.ShapeDtypeStruct(shape, _f32)


def kernel(x, edge_index, W1, b1, W2, b2, W3, b3, Wc, bc):
    src = edge_index[0]
    dst = edge_index[1]
    pad = EPT * NW - E
    srcf = jnp.concatenate([src, jnp.zeros((pad,), src.dtype)])
    dstf = jnp.concatenate([dst, jnp.full((pad,), N, dst.dtype)])
    dstp = dstf.reshape(NW, CHUNKS, CHUNK)

    def _split(flat):
        n0 = NS * K0 * CHUNK
        a = flat[:n0].reshape(NS, K0, CHUNK)
        b = flat[n0:].reshape(NS, K1, CHUNK)
        b = jnp.pad(b, ((0, 0), (0, K0 - K1), (0, 0)))
        return jnp.concatenate([a, b], axis=0)

    srcp_a = _split(srcf)
    dstp_a = _split(dstf)
    ones = jnp.ones((CHUNK,), _f32)
    z1 = jnp.zeros((NPAD,), _f32)
    z64 = jnp.zeros((NPAD, H), _f32)
    z32 = jnp.zeros((NPAD, H3), _f32)

    degp = _deg_kernel(dstp, ones, z1)
    dinv, hs1 = pl.pallas_call(
        _mm_dinv_body, out_shape=(_sds((N, 1)), _sds((N, H))))(
        x, W1, degp[0, :N, None], degp[1, :N, None])

    agg1 = _agg64(srcp_a, dstp_a, hs1, z64)
    hs2 = pl.pallas_call(_mid_body, out_shape=_sds((N, H)))(
        agg1[0, :N], agg1[1, :N], hs1, dinv, b1.reshape(1, H), W2)

    agg2 = _agg64(srcp_a, dstp_a, hs2, z64)
    hs3 = pl.pallas_call(_mid_body, out_shape=_sds((N, H3)))(
        agg2[0, :N], agg2[1, :N], hs2, dinv, b2.reshape(1, H), W3)

    agg3 = _agg32(srcp_a, dstp_a, hs3, z32)
    h3, z = pl.pallas_call(_final_body, out_shape=(_sds((N, H3)), _sds((N, C))))(
        agg3[0, :N], agg3[1, :N], hs3, dinv, b3.reshape(1, H3), Wc, bc.reshape(1, C))
    return (h3, z)


# speedup vs baseline: 33.8249x; 1.0265x over previous
import functools

import jax
import jax.numpy as jnp
from jax import lax
from jax.experimental import pallas as pl
from jax.experimental.pallas import tpu as pltpu
from jax.experimental.pallas import tpu_sc as plsc

N = 10000
E = 320000
D = 128
H = 64
H3 = 32
C = 4

NC = 2
NS = 16
NW = NC * NS
CHUNK = 128
NCHUNKS = E // CHUNK
K = 80
NCPAD = NW * K
NPAD = 10240
RPT = NPAD // NS
NRT = N // NS

_f32 = jnp.float32


def _sc_mesh():
    return plsc.VectorSubcoreMesh(core_axis_name="c", subcore_axis_name="s")


@functools.partial(
    pl.kernel,
    out_type=jax.ShapeDtypeStruct((NC, NPAD), _f32),
    mesh=_sc_mesh(),
    scratch_types=[
        pltpu.VMEM((K, CHUNK), jnp.int32),
        pltpu.VMEM((CHUNK,), _f32),
        pltpu.VMEM_SHARED((NPAD,), _f32),
    ],
)
def _deg_kernel(dst3_hbm, ones_hbm, zeros_hbm, out_hbm, dst_v, ones_v, deg_sh):
    c = lax.axis_index("c")
    s = lax.axis_index("s")
    wid = c * NS + s
    pltpu.sync_copy(zeros_hbm.at[pl.ds(s * RPT, RPT)], deg_sh.at[pl.ds(s * RPT, RPT)])
    pltpu.sync_copy(dst3_hbm.at[wid], dst_v)
    pltpu.sync_copy(ones_hbm, ones_v)
    plsc.subcore_barrier()

    def body(j, carry):
        pltpu.sync_copy(ones_v, deg_sh.at[dst_v.at[j]], add=True)
        return carry

    lax.fori_loop(0, K, body, 0)
    plsc.subcore_barrier()
    pltpu.sync_copy(deg_sh.at[pl.ds(s * RPT, RPT)], out_hbm.at[c, pl.ds(s * RPT, RPT)])


def _make_agg_kernel(h):
    @functools.partial(
        pl.kernel,
        out_type=jax.ShapeDtypeStruct((NC, NPAD, h), _f32),
        mesh=_sc_mesh(),
        compiler_params=pltpu.CompilerParams(use_tc_tiling_on_sc=False),
        scratch_types=[
            pltpu.VMEM((K, CHUNK), jnp.int32),
            pltpu.VMEM((K, CHUNK), jnp.int32),
            pltpu.VMEM((CHUNK, h), _f32),
            pltpu.VMEM((CHUNK, h), _f32),
            pltpu.VMEM_SHARED((NPAD, h), _f32),
            pltpu.VMEM_SHARED((N, h), _f32),
            pltpu.SemaphoreType.DMA,
            pltpu.SemaphoreType.DMA,
        ],
    )
    def agg_kernel(src3_hbm, dst3_hbm, hs_hbm, zeros_hbm, out_hbm,
                   src_v, dst_v, msg0_v, msg1_v, agg_sh, hs_sh, sem0, sem1):
        c = lax.axis_index("c")
        s = lax.axis_index("s")
        wid = c * NS + s
        pltpu.sync_copy(zeros_hbm.at[pl.ds(s * RPT, RPT)],
                        agg_sh.at[pl.ds(s * RPT, RPT)])
        pltpu.sync_copy(hs_hbm.at[pl.ds(s * NRT, NRT)],
                        hs_sh.at[pl.ds(s * NRT, NRT)])
        pltpu.sync_copy(src3_hbm.at[wid], src_v)
        pltpu.sync_copy(dst3_hbm.at[wid], dst_v)
        plsc.subcore_barrier()

        def run_pipe(nchunks):
            pltpu.async_copy(hs_sh.at[src_v.at[0]], msg0_v, sem0)

            def body(i, carry):
                j = 2 * i
                pltpu.async_copy(hs_sh.at[src_v.at[j + 1]], msg1_v, sem1)
                pltpu.make_async_copy(hs_sh.at[src_v.at[j]], msg0_v, sem0).wait()
                pltpu.sync_copy(msg0_v, agg_sh.at[dst_v.at[j]], add=True)

                @pl.when(i < nchunks // 2 - 1)
                def _():
                    pltpu.async_copy(hs_sh.at[src_v.at[j + 2]], msg0_v, sem0)

                pltpu.make_async_copy(hs_sh.at[src_v.at[j + 1]], msg1_v, sem1).wait()
                pltpu.sync_copy(msg1_v, agg_sh.at[dst_v.at[j + 1]], add=True)
                return carry

            lax.fori_loop(0, nchunks // 2, body, 0)

        run_pipe(K)
        plsc.subcore_barrier()
        pltpu.sync_copy(agg_sh.at[pl.ds(s * RPT, RPT)],
                        out_hbm.at[c, pl.ds(s * RPT, RPT)])

    return agg_kernel


_agg64 = _make_agg_kernel(H)
_agg32 = _make_agg_kernel(H3)


def _mm_dinv_body(x_ref, w_ref, dega_ref, degb_ref, dinv_ref, hs_ref):
    deg = dega_ref[...] + degb_ref[...] + 1.0
    dinv = lax.rsqrt(deg)
    dinv_ref[...] = dinv
    h = jnp.dot(x_ref[...], w_ref[...], preferred_element_type=_f32)
    hs_ref[...] = h * dinv


def _mid_body(agga_ref, aggb_ref, hs_ref, dinv_ref, b_ref, w_ref, o_ref):
    dinv = dinv_ref[...]
    hcur = jnp.maximum(
        dinv * (agga_ref[...] + aggb_ref[...] + hs_ref[...]) + b_ref[...], 0.0)
    o_ref[...] = jnp.dot(hcur, w_ref[...], preferred_element_type=_f32) * dinv


def _final_body(agga_ref, aggb_ref, hs_ref, dinv_ref, b_ref, wc_ref, bc_ref,
                h3_ref, z_ref):
    dinv = dinv_ref[...]
    h3 = jnp.maximum(
        dinv * (agga_ref[...] + aggb_ref[...] + hs_ref[...]) + b_ref[...], 0.0)
    h3_ref[...] = h3
    z_ref[...] = jnp.dot(h3, wc_ref[...], preferred_element_type=_f32) + bc_ref[...]


def _sds(shape):
    return jax.ShapeDtypeStruct(shape, _f32)


def kernel(x, edge_index, W1, b1, W2, b2, W3, b3, Wc, bc):
    ei3 = edge_index.reshape(2, NCHUNKS, CHUNK)
    padc = NCPAD - NCHUNKS
    src3 = jnp.concatenate(
        [ei3[0], jnp.zeros((padc, CHUNK), edge_index.dtype)]).reshape(NW, K, CHUNK)
    dst3 = jnp.concatenate(
        [ei3[1], jnp.full((padc, CHUNK), N, edge_index.dtype)]).reshape(NW, K, CHUNK)
    ones = jnp.ones((CHUNK,), _f32)
    z1 = jnp.zeros((NPAD,), _f32)
    z64 = jnp.zeros((NPAD, H), _f32)
    z32 = jnp.zeros((NPAD, H3), _f32)

    degp = _deg_kernel(dst3, ones, z1)
    dinv, hs1 = pl.pallas_call(
        _mm_dinv_body, out_shape=(_sds((N, 1)), _sds((N, H))))(
        x, W1, degp[0, :N, None], degp[1, :N, None])

    agg1 = _agg64(src3, dst3, hs1, z64)
    hs2 = pl.pallas_call(_mid_body, out_shape=_sds((N, H)))(
        agg1[0, :N], agg1[1, :N], hs1, dinv, b1.reshape(1, H), W2)

    agg2 = _agg64(src3, dst3, hs2, z64)
    hs3 = pl.pallas_call(_mid_body, out_shape=_sds((N, H3)))(
        agg2[0, :N], agg2[1, :N], hs2, dinv, b2.reshape(1, H), W3)

    agg3 = _agg32(src3, dst3, hs3, z32)
    h3, z = pl.pallas_call(_final_body, out_shape=(_sds((N, H3)), _sds((N, C))))(
        agg3[0, :N], agg3[1, :N], hs3, dinv, b3.reshape(1, H3), Wc, bc.reshape(1, C))
    return (h3, z)


# speedup vs baseline: 35.9290x; 1.0622x over previous
import functools

import jax
import jax.numpy as jnp
from jax import lax
from jax.experimental import pallas as pl
from jax.experimental.pallas import tpu as pltpu
from jax.experimental.pallas import tpu_sc as plsc

N = 10000
E = 320000
D = 128
H = 64
H3 = 32
C = 4

NC = 2
NS = 16
NW = NC * NS
CHUNK = 128
NCHUNKS = E // CHUNK
K = 80
NCPAD = NW * K
NPAD = 10240
RPT = NPAD // NS
NRT = N // NS

_f32 = jnp.float32


def _sc_mesh():
    return plsc.VectorSubcoreMesh(core_axis_name="c", subcore_axis_name="s")


@functools.partial(
    pl.kernel,
    out_type=jax.ShapeDtypeStruct((NC, NPAD), _f32),
    mesh=_sc_mesh(),
    scratch_types=[
        pltpu.VMEM((K, CHUNK), jnp.int32),
        pltpu.VMEM((CHUNK,), _f32),
        pltpu.VMEM_SHARED((NPAD,), _f32),
    ],
)
def _deg_kernel(dst3_hbm, ones_hbm, zeros_hbm, out_hbm, dst_v, ones_v, deg_sh):
    c = lax.axis_index("c")
    s = lax.axis_index("s")
    wid = c * NS + s
    pltpu.sync_copy(zeros_hbm.at[pl.ds(s * RPT, RPT)], deg_sh.at[pl.ds(s * RPT, RPT)])
    pltpu.sync_copy(dst3_hbm.at[wid], dst_v)
    pltpu.sync_copy(ones_hbm, ones_v)
    plsc.subcore_barrier()

    def body(j, carry):
        pltpu.sync_copy(ones_v, deg_sh.at[dst_v.at[j]], add=True)
        return carry

    lax.fori_loop(0, K, body, 0)
    plsc.subcore_barrier()
    pltpu.sync_copy(deg_sh.at[pl.ds(s * RPT, RPT)], out_hbm.at[c, pl.ds(s * RPT, RPT)])


def _make_agg_kernel(h):
    @functools.partial(
        pl.kernel,
        out_type=jax.ShapeDtypeStruct((NC, NPAD, h), _f32),
        mesh=_sc_mesh(),
        compiler_params=pltpu.CompilerParams(use_tc_tiling_on_sc=False),
        scratch_types=[
            pltpu.VMEM((K, CHUNK), jnp.int32),
            pltpu.VMEM((K, CHUNK), jnp.int32),
            pltpu.VMEM((CHUNK, h), _f32),
            pltpu.VMEM((CHUNK, h), _f32),
            pltpu.VMEM_SHARED((NPAD, h), _f32),
            pltpu.VMEM_SHARED((N, h), _f32),
            pltpu.SemaphoreType.DMA,
            pltpu.SemaphoreType.DMA,
        ],
    )
    def agg_kernel(src3_hbm, dst3_hbm, hs_hbm, zeros_hbm, out_hbm,
                   src_v, dst_v, msg0_v, msg1_v, agg_sh, hs_sh, sem0, sem1):
        c = lax.axis_index("c")
        s = lax.axis_index("s")
        wid = c * NS + s
        pltpu.sync_copy(zeros_hbm.at[pl.ds(s * RPT, RPT)],
                        agg_sh.at[pl.ds(s * RPT, RPT)])
        pltpu.sync_copy(hs_hbm.at[pl.ds(s * NRT, NRT)],
                        hs_sh.at[pl.ds(s * NRT, NRT)])
        pltpu.sync_copy(src3_hbm.at[wid], src_v)
        pltpu.sync_copy(dst3_hbm.at[wid], dst_v)
        plsc.subcore_barrier()

        def run_pipe(nchunks):
            pltpu.async_copy(hs_sh.at[src_v.at[0]], msg0_v, sem0)

            def body(i, carry):
                j = 2 * i
                pltpu.async_copy(hs_sh.at[src_v.at[j + 1]], msg1_v, sem1)
                pltpu.make_async_copy(hs_sh.at[src_v.at[j]], msg0_v, sem0).wait()
                pltpu.sync_copy(msg0_v, agg_sh.at[dst_v.at[j]], add=True)

                @pl.when(i < nchunks // 2 - 1)
                def _():
                    pltpu.async_copy(hs_sh.at[src_v.at[j + 2]], msg0_v, sem0)

                pltpu.make_async_copy(hs_sh.at[src_v.at[j + 1]], msg1_v, sem1).wait()
                pltpu.sync_copy(msg1_v, agg_sh.at[dst_v.at[j + 1]], add=True)
                return carry

            lax.fori_loop(0, nchunks // 2, body, 0)

        run_pipe(K)
        plsc.subcore_barrier()
        pltpu.sync_copy(agg_sh.at[pl.ds(s * RPT, RPT)],
                        out_hbm.at[c, pl.ds(s * RPT, RPT)])

    return agg_kernel


_agg64 = _make_agg_kernel(H)
_agg32 = _make_agg_kernel(H3)


def _mm_dinv_body(x_ref, w_ref, dega_ref, degb_ref, dinv_ref, hs_ref):
    deg = dega_ref[...] + degb_ref[...] + 1.0
    dinv = lax.rsqrt(deg)
    dinv_ref[...] = dinv
    h = jnp.dot(x_ref[...], w_ref[...], preferred_element_type=_f32)
    hs_ref[...] = h * dinv


def _mid_body(aggp_ref, hs_ref, dinv_ref, b_ref, w_ref, o_ref):
    dinv = dinv_ref[...]
    hcur = jnp.maximum(
        dinv * (aggp_ref[0, :N, :] + aggp_ref[1, :N, :] + hs_ref[...])
        + b_ref[...], 0.0)
    o_ref[...] = jnp.dot(hcur, w_ref[...], preferred_element_type=_f32) * dinv


def _final_body(aggp_ref, hs_ref, dinv_ref, b_ref, wc_ref, bc_ref,
                h3_ref, z_ref):
    dinv = dinv_ref[...]
    h3 = jnp.maximum(
        dinv * (aggp_ref[0, :N, :] + aggp_ref[1, :N, :] + hs_ref[...])
        + b_ref[...], 0.0)
    h3_ref[...] = h3
    z_ref[...] = jnp.dot(h3, wc_ref[...], preferred_element_type=_f32) + bc_ref[...]


def _sds(shape):
    return jax.ShapeDtypeStruct(shape, _f32)


def kernel(x, edge_index, W1, b1, W2, b2, W3, b3, Wc, bc):
    ei3 = edge_index.reshape(2, NCHUNKS, CHUNK)
    padc = NCPAD - NCHUNKS
    src3 = jnp.concatenate(
        [ei3[0], jnp.zeros((padc, CHUNK), edge_index.dtype)]).reshape(NW, K, CHUNK)
    dst3 = jnp.concatenate(
        [ei3[1], jnp.full((padc, CHUNK), N, edge_index.dtype)]).reshape(NW, K, CHUNK)
    ones = jnp.ones((CHUNK,), _f32)
    z1 = jnp.zeros((NPAD,), _f32)
    z64 = jnp.zeros((NPAD, H), _f32)
    z32 = jnp.zeros((NPAD, H3), _f32)

    degp = _deg_kernel(dst3, ones, z1)
    dinv, hs1 = pl.pallas_call(
        _mm_dinv_body, out_shape=(_sds((N, 1)), _sds((N, H))))(
        x, W1, degp[0, :N, None], degp[1, :N, None])

    agg1 = _agg64(src3, dst3, hs1, z64)
    hs2 = pl.pallas_call(_mid_body, out_shape=_sds((N, H)))(
        agg1, hs1, dinv, b1.reshape(1, H), W2)

    agg2 = _agg64(src3, dst3, hs2, z64)
    hs3 = pl.pallas_call(_mid_body, out_shape=_sds((N, H3)))(
        agg2, hs2, dinv, b2.reshape(1, H), W3)

    agg3 = _agg32(src3, dst3, hs3, z32)
    h3, z = pl.pallas_call(_final_body, out_shape=(_sds((N, H3)), _sds((N, C))))(
        agg3, hs3, dinv, b3.reshape(1, H3), Wc, bc.reshape(1, C))
    return (h3, z)
